# Initial kernel scaffold; baseline (speedup 1.0000x reference)
import jax
import jax.numpy as jnp
from jax.experimental import pallas as pl


def kernel(x, fc1_w, fc1_b, sage1_wl, sage1_bl, sage1_wr, bn1_g, bn1_b, sage2_wl, sage2_bl, sage2_wr, bn2_g, bn2_b, sage3_wl, sage3_bl, sage3_wr, bn3_g, bn3_b, fc2_w, fc2_b, bn4_g, bn4_b, fc3_w, fc3_b, bn5_g, bn5_b, fc5_w, fc5_b, fc4_w, fc4_b, edge_index, batch):
    raise NotImplementedError("write your pallas kernel here")


# speedup vs baseline: 2.6736x; 2.6736x over previous
import jax
import jax.numpy as jnp
from jax import lax
from jax.experimental import pallas as pl
from jax.experimental.pallas import tpu as pltpu
from jax.experimental.pallas import tpu_sc as plsc

N = 10000
E = 160000
G = 64
EPS = 1e-5
ROWS = 1000
NB = N // ROWS

NSUB = 16
NW = 2 * NSUB
CHUNK = 128
CPW = 40
EPAD = NW * CPW * CHUNK
RPT = 624
RPT_LAST = N - 15 * RPT
NACC = N + 8
DEGW = 128
DINVW = 16


def _rows_spec(cols):
    return pl.BlockSpec((ROWS, cols), lambda i: (i, 0))


def _full_spec(shape):
    nd = len(shape)
    return pl.BlockSpec(shape, lambda i: (0,) * nd)


def _sliced_copy(src, dst, s):
    lo = s * RPT

    @pl.when(s < NSUB - 1)
    def _():
        pltpu.sync_copy(src.at[pl.ds(lo, RPT)], dst.at[pl.ds(lo, RPT)])

    @pl.when(s == NSUB - 1)
    def _():
        pltpu.sync_copy(src.at[pl.ds(15 * RPT, RPT_LAST)],
                        dst.at[pl.ds(15 * RPT, RPT_LAST)])


def _make_sc_agg(d):
    mesh = plsc.VectorSubcoreMesh(core_axis_name="c", subcore_axis_name="s")

    def body(y, srcs, dsts, zer, agg_out, srcv, dstv, buf, acc):
        c = lax.axis_index("c")
        s = lax.axis_index("s")
        w = c * NSUB + s
        _sliced_copy(zer, acc, s)
        pltpu.sync_copy(srcs.at[w], srcv)
        pltpu.sync_copy(dsts.at[w], dstv)
        plsc.subcore_barrier()

        def step(j, carry):
            pltpu.sync_copy(y.at[srcv.at[j]], buf)
            pltpu.sync_copy(buf, acc.at[dstv.at[j]], add=True)
            return carry

        lax.fori_loop(0, CPW, step, 0)
        plsc.subcore_barrier()
        _sliced_copy(acc, agg_out.at[c], s)

    return pl.kernel(
        body,
        out_type=jax.ShapeDtypeStruct((2, N, d), jnp.float32),
        mesh=mesh,
        scratch_types=[
            pltpu.VMEM((CPW, CHUNK), jnp.int32),
            pltpu.VMEM((CPW, CHUNK), jnp.int32),
            pltpu.VMEM((CHUNK, d), jnp.float32),
            pltpu.VMEM_SHARED((NACC, d), jnp.float32),
        ])


def _make_sc_deg():
    mesh = plsc.VectorSubcoreMesh(core_axis_name="c", subcore_axis_name="s")

    def body(dsts, zerdeg, ones, deg_out, dstv, onev, dacc):
        c = lax.axis_index("c")
        s = lax.axis_index("s")
        w = c * NSUB + s
        _sliced_copy(zerdeg, dacc, s)
        pltpu.sync_copy(dsts.at[w], dstv)
        pltpu.sync_copy(ones, onev)
        plsc.subcore_barrier()

        def step(j, carry):
            pltpu.sync_copy(onev, dacc.at[dstv.at[j]], add=True)
            return carry

        lax.fori_loop(0, CPW, step, 0)
        plsc.subcore_barrier()
        _sliced_copy(dacc, deg_out.at[c], s)

    return pl.kernel(
        body,
        out_type=jax.ShapeDtypeStruct((2, N, DEGW), jnp.float32),
        mesh=mesh,
        scratch_types=[
            pltpu.VMEM((CPW, CHUNK), jnp.int32),
            pltpu.VMEM((CHUNK, DEGW), jnp.float32),
            pltpu.VMEM_SHARED((NACC, DEGW), jnp.float32),
        ])


def _c1_body(x, w, b, o):
    o[...] = jnp.maximum(
        jnp.dot(x[...], w[...], preferred_element_type=jnp.float32) + b[...],
        0.0)


def _c2_body(agg, deg, h0, wl, bl, wr, z_ref, dinv_ref, ssum, ssq, a1, a2):
    i = pl.program_id(0)
    aggs = agg[0] + agg[1]
    degs = deg[0, :, 0:1] + deg[1, :, 0:1]
    dinv = 1.0 / jnp.maximum(degs, 1.0)
    dinv_ref[...] = jnp.broadcast_to(dinv, (dinv.shape[0], DINVW))
    z = (jnp.dot(aggs * dinv, wl[...], preferred_element_type=jnp.float32)
         + bl[...]
         + jnp.dot(h0[...], wr[...], preferred_element_type=jnp.float32))
    z_ref[...] = z

    @pl.when(i == 0)
    def _():
        a1[...] = jnp.zeros_like(a1)
        a2[...] = jnp.zeros_like(a2)

    a1[...] += jnp.sum(z, axis=0, keepdims=True)
    a2[...] += jnp.sum(z * z, axis=0, keepdims=True)

    @pl.when(i == NB - 1)
    def _():
        ssum[...] = a1[...]
        ssq[...] = a2[...]


def _bn2mm_body(z, s, q, g, b, wl, wr, y_ref, r_ref):
    mu = s[...] * (1.0 / N)
    var = jnp.maximum(q[...] * (1.0 / N) - mu * mu, 0.0)
    rstd = lax.rsqrt(var + EPS)
    h = jnp.maximum((z[...] - mu) * rstd * g[...] + b[...], 0.0)
    y_ref[...] = jnp.dot(h, wl[...], preferred_element_type=jnp.float32)
    r_ref[...] = jnp.dot(h, wr[...], preferred_element_type=jnp.float32)


def _aggadd_body(agg, dinv, r, bl, z_ref, ssum, ssq, a1, a2):
    i = pl.program_id(0)
    z = (agg[0] + agg[1])[:, :r.shape[-1]] * dinv[:, 0:1] + bl[...] + r[...]
    z_ref[...] = z

    @pl.when(i == 0)
    def _():
        a1[...] = jnp.zeros_like(a1)
        a2[...] = jnp.zeros_like(a2)

    a1[...] += jnp.sum(z, axis=0, keepdims=True)
    a2[...] += jnp.sum(z * z, axis=0, keepdims=True)

    @pl.when(i == NB - 1)
    def _():
        ssum[...] = a1[...]
        ssq[...] = a2[...]


def _bnmm_body(z, s, q, g, b, w, wb, z2_ref, osum, osq, a1, a2):
    i = pl.program_id(0)
    mu = s[...] * (1.0 / N)
    var = jnp.maximum(q[...] * (1.0 / N) - mu * mu, 0.0)
    h = jnp.maximum((z[...] - mu) * lax.rsqrt(var + EPS) * g[...] + b[...],
                    0.0)
    z2 = jnp.dot(h, w[...], preferred_element_type=jnp.float32) + wb[...]
    z2_ref[...] = z2

    @pl.when(i == 0)
    def _():
        a1[...] = jnp.zeros_like(a1)
        a2[...] = jnp.zeros_like(a2)

    a1[...] += jnp.sum(z2, axis=0, keepdims=True)
    a2[...] += jnp.sum(z2 * z2, axis=0, keepdims=True)

    @pl.when(i == NB - 1)
    def _():
        osum[...] = a1[...]
        osq[...] = a2[...]


def _c8_body(z, s, q, g, b, w, wb, bt, z2_ref, osum, osq, sums_ref, cnts_ref,
             a1, a2, asu, acn):
    i = pl.program_id(0)
    mu = s[...] * (1.0 / N)
    var = jnp.maximum(q[...] * (1.0 / N) - mu * mu, 0.0)
    h = jnp.maximum((z[...] - mu) * lax.rsqrt(var + EPS) * g[...] + b[...],
                    0.0)
    z2 = jnp.dot(h, w[...], preferred_element_type=jnp.float32) + wb[...]
    z2_ref[...] = z2
    oh = (bt[...] == lax.broadcasted_iota(jnp.int32, (1, G), 1)).astype(
        jnp.float32)

    @pl.when(i == 0)
    def _():
        a1[...] = jnp.zeros_like(a1)
        a2[...] = jnp.zeros_like(a2)
        asu[...] = jnp.zeros_like(asu)
        acn[...] = jnp.zeros_like(acn)

    a1[...] += jnp.sum(z2, axis=0, keepdims=True)
    a2[...] += jnp.sum(z2 * z2, axis=0, keepdims=True)
    asu[...] += lax.dot_general(oh, h, (((0,), (0,)), ((), ())),
                                preferred_element_type=jnp.float32)
    acn[...] += lax.dot_general(oh, jnp.ones((ROWS, 8), jnp.float32),
                                (((0,), (0,)), ((), ())),
                                preferred_element_type=jnp.float32)

    @pl.when(i == NB - 1)
    def _():
        osum[...] = a1[...]
        osq[...] = a2[...]
        sums_ref[...] = asu[...]
        cnts_ref[...] = acn[...]


def _c9_body(z, s, q, g, b, w5, b5, sums, cnts, w4, b4, nout, gout):
    i = pl.program_id(0)
    mu = s[...] * (1.0 / N)
    var = jnp.maximum(q[...] * (1.0 / N) - mu * mu, 0.0)
    h = jnp.maximum((z[...] - mu) * lax.rsqrt(var + EPS) * g[...] + b[...],
                    0.0)
    nout[...] = jnp.dot(h, w5[...], preferred_element_type=jnp.float32) \
        + b5[...]

    @pl.when(i == 0)
    def _():
        gv = sums[...] * (1.0 / jnp.maximum(cnts[...][:, 0:1], 1.0))
        gout[...] = jnp.dot(gv, w4[...],
                            preferred_element_type=jnp.float32) + b4[...]


def kernel(x, fc1_w, fc1_b, sage1_wl, sage1_bl, sage1_wr, bn1_g, bn1_b,
           sage2_wl, sage2_bl, sage2_wr, bn2_g, bn2_b,
           sage3_wl, sage3_bl, sage3_wr, bn3_g, bn3_b,
           fc2_w, fc2_b, bn4_g, bn4_b, fc3_w, fc3_b, bn5_g, bn5_b,
           fc5_w, fc5_b, fc4_w, fc4_b, edge_index, batch):
    f32 = jnp.float32
    row2 = lambda v: v.reshape(1, -1)

    pad = EPAD - E
    src = jnp.concatenate(
        [edge_index[0], jnp.zeros((pad,), jnp.int32)]).reshape(NW, CPW, CHUNK)
    dst = jnp.concatenate(
        [edge_index[1],
         N + (jnp.arange(pad, dtype=jnp.int32) % 8)]).reshape(NW, CPW, CHUNK)
    zer128 = jnp.zeros((N, 128), f32)
    onesdeg = jnp.ones((CHUNK, DEGW), f32)
    batch2 = batch.reshape(N, 1)

    h0 = pl.pallas_call(
        _c1_body, grid=(NB,),
        in_specs=[_rows_spec(256), _full_spec((256, 128)),
                  _full_spec((1, 128))],
        out_specs=_rows_spec(128),
        out_shape=jax.ShapeDtypeStruct((N, 128), f32),
    )(x, fc1_w.T, row2(fc1_b))

    deg = _make_sc_deg()(dst, zer128, onesdeg)
    agg1 = _make_sc_agg(128)(h0, src, dst, zer128)

    z1, dinv, s1, q1 = pl.pallas_call(
        _c2_body, grid=(NB,),
        in_specs=[pl.BlockSpec((2, ROWS, 128), lambda i: (0, i, 0)),
                  pl.BlockSpec((2, ROWS, DEGW), lambda i: (0, i, 0)),
                  _rows_spec(128), _full_spec((128, 256)),
                  _full_spec((1, 256)), _full_spec((128, 256))],
        out_specs=[_rows_spec(256), _rows_spec(DINVW),
                   _full_spec((1, 256)), _full_spec((1, 256))],
        out_shape=[jax.ShapeDtypeStruct((N, 256), f32),
                   jax.ShapeDtypeStruct((N, DINVW), f32),
                   jax.ShapeDtypeStruct((1, 256), f32),
                   jax.ShapeDtypeStruct((1, 256), f32)],
        scratch_shapes=[pltpu.VMEM((1, 256), f32), pltpu.VMEM((1, 256), f32)],
    )(agg1, deg, h0, sage1_wl.T, row2(sage1_bl), sage1_wr.T)

    y2, r2 = pl.pallas_call(
        _bn2mm_body, grid=(NB,),
        in_specs=[_rows_spec(256), _full_spec((1, 256)), _full_spec((1, 256)),
                  _full_spec((1, 256)), _full_spec((1, 256)),
                  _full_spec((256, 128)), _full_spec((256, 128))],
        out_specs=[_rows_spec(128), _rows_spec(128)],
        out_shape=[jax.ShapeDtypeStruct((N, 128), f32),
                   jax.ShapeDtypeStruct((N, 128), f32)],
    )(z1, s1, q1, row2(bn1_g), row2(bn1_b), sage2_wl.T, sage2_wr.T)

    agg2 = _make_sc_agg(128)(y2, src, dst, zer128)

    z2, s2, q2 = pl.pallas_call(
        _aggadd_body, grid=(NB,),
        in_specs=[pl.BlockSpec((2, ROWS, 128), lambda i: (0, i, 0)),
                  _rows_spec(DINVW), _rows_spec(128), _full_spec((1, 128))],
        out_specs=[_rows_spec(128), _full_spec((1, 128)),
                   _full_spec((1, 128))],
        out_shape=[jax.ShapeDtypeStruct((N, 128), f32),
                   jax.ShapeDtypeStruct((1, 128), f32),
                   jax.ShapeDtypeStruct((1, 128), f32)],
        scratch_shapes=[pltpu.VMEM((1, 128), f32), pltpu.VMEM((1, 128), f32)],
    )(agg2, dinv, r2, row2(sage2_bl))

    wl3p = jnp.zeros((128, 128), f32).at[:, :64].set(sage3_wl.T)
    y3, r3 = pl.pallas_call(
        _bn2mm_body, grid=(NB,),
        in_specs=[_rows_spec(128), _full_spec((1, 128)), _full_spec((1, 128)),
                  _full_spec((1, 128)), _full_spec((1, 128)),
                  _full_spec((128, 128)), _full_spec((128, 64))],
        out_specs=[_rows_spec(128), _rows_spec(64)],
        out_shape=[jax.ShapeDtypeStruct((N, 128), f32),
                   jax.ShapeDtypeStruct((N, 64), f32)],
    )(z2, s2, q2, row2(bn2_g), row2(bn2_b), wl3p, sage3_wr.T)

    agg3 = _make_sc_agg(128)(y3, src, dst, zer128)

    z3, s3, q3 = pl.pallas_call(
        _aggadd_body, grid=(NB,),
        in_specs=[pl.BlockSpec((2, ROWS, 128), lambda i: (0, i, 0)),
                  _rows_spec(DINVW), _rows_spec(64), _full_spec((1, 64))],
        out_specs=[_rows_spec(64), _full_spec((1, 64)), _full_spec((1, 64))],
        out_shape=[jax.ShapeDtypeStruct((N, 64), f32),
                   jax.ShapeDtypeStruct((1, 64), f32),
                   jax.ShapeDtypeStruct((1, 64), f32)],
        scratch_shapes=[pltpu.VMEM((1, 64), f32), pltpu.VMEM((1, 64), f32)],
    )(agg3, dinv, r3, row2(sage3_bl))

    z4, s4, q4 = pl.pallas_call(
        _bnmm_body, grid=(NB,),
        in_specs=[_rows_spec(64), _full_spec((1, 64)), _full_spec((1, 64)),
                  _full_spec((1, 64)), _full_spec((1, 64)),
                  _full_spec((64, 64)), _full_spec((1, 64))],
        out_specs=[_rows_spec(64), _full_spec((1, 64)), _full_spec((1, 64))],
        out_shape=[jax.ShapeDtypeStruct((N, 64), f32),
                   jax.ShapeDtypeStruct((1, 64), f32),
                   jax.ShapeDtypeStruct((1, 64), f32)],
        scratch_shapes=[pltpu.VMEM((1, 64), f32), pltpu.VMEM((1, 64), f32)],
    )(z3, s3, q3, row2(bn3_g), row2(bn3_b), fc2_w.T, row2(fc2_b))

    z5, s5, q5, sums, cnts = pl.pallas_call(
        _c8_body, grid=(NB,),
        in_specs=[_rows_spec(64), _full_spec((1, 64)), _full_spec((1, 64)),
                  _full_spec((1, 64)), _full_spec((1, 64)),
                  _full_spec((64, 128)), _full_spec((1, 128)),
                  _rows_spec(1)],
        out_specs=[_rows_spec(128), _full_spec((1, 128)),
                   _full_spec((1, 128)), _full_spec((G, 64)),
                   _full_spec((G, 8))],
        out_shape=[jax.ShapeDtypeStruct((N, 128), f32),
                   jax.ShapeDtypeStruct((1, 128), f32),
                   jax.ShapeDtypeStruct((1, 128), f32),
                   jax.ShapeDtypeStruct((G, 64), f32),
                   jax.ShapeDtypeStruct((G, 8), f32)],
        scratch_shapes=[pltpu.VMEM((1, 128), f32), pltpu.VMEM((1, 128), f32),
                        pltpu.VMEM((G, 64), f32), pltpu.VMEM((G, 8), f32)],
    )(z4, s4, q4, row2(bn4_g), row2(bn4_b), fc3_w.T, row2(fc3_b), batch2)

    w5p = jnp.zeros((128, 128), f32).at[:, :5].set(fc5_w.T)
    b5p = jnp.zeros((1, 128), f32).at[0, :5].set(fc5_b)
    w4p = jnp.zeros((64, 128), f32).at[:, :3].set(fc4_w.T)
    b4p = jnp.zeros((1, 128), f32).at[0, :3].set(fc4_b)
    noutp, goutp = pl.pallas_call(
        _c9_body, grid=(NB,),
        in_specs=[_rows_spec(128), _full_spec((1, 128)), _full_spec((1, 128)),
                  _full_spec((1, 128)), _full_spec((1, 128)),
                  _full_spec((128, 128)), _full_spec((1, 128)),
                  _full_spec((G, 64)), _full_spec((G, 8)),
                  _full_spec((64, 128)), _full_spec((1, 128))],
        out_specs=[_rows_spec(128), _full_spec((G, 128))],
        out_shape=[jax.ShapeDtypeStruct((N, 128), f32),
                   jax.ShapeDtypeStruct((G, 128), f32)],
    )(z5, s5, q5, row2(bn5_g), row2(bn5_b), w5p, b5p, sums, cnts, w4p, b4p)

    return noutp[:, :5], goutp[:, :3]


# speedup vs baseline: 2.9306x; 1.0961x over previous
import jax
import jax.numpy as jnp
from jax import lax
from jax.experimental import pallas as pl
from jax.experimental.pallas import tpu as pltpu
from jax.experimental.pallas import tpu_sc as plsc

N = 10000
E = 160000
G = 64
EPS = 1e-5
ROWS = 1000
NB = N // ROWS

NSUB = 16
NW = 2 * NSUB
CHUNK = 128
CPW = 40
EPAD = NW * CPW * CHUNK
RPT = 624
RPT_LAST = N - 15 * RPT
NACC = N + 8
DEGW = 128
DINVW = 16


def _rows_spec(cols):
    return pl.BlockSpec((ROWS, cols), lambda i: (i, 0))


def _full_spec(shape):
    nd = len(shape)
    return pl.BlockSpec(shape, lambda i: (0,) * nd)


def _sliced_copy(src, dst, s):
    lo = s * RPT

    @pl.when(s < NSUB - 1)
    def _():
        pltpu.sync_copy(src.at[pl.ds(lo, RPT)], dst.at[pl.ds(lo, RPT)])

    @pl.when(s == NSUB - 1)
    def _():
        pltpu.sync_copy(src.at[pl.ds(15 * RPT, RPT_LAST)],
                        dst.at[pl.ds(15 * RPT, RPT_LAST)])


def _make_sc_agg(d):
    mesh = plsc.VectorSubcoreMesh(core_axis_name="c", subcore_axis_name="s")

    def body(y, srcs, dsts, zer, agg_out, srcv, dstv, bufa, bufb, acc,
             sga, sgb):
        c = lax.axis_index("c")
        s = lax.axis_index("s")
        w = c * NSUB + s
        _sliced_copy(zer, acc, s)
        pltpu.sync_copy(srcs.at[w], srcv)
        pltpu.sync_copy(dsts.at[w], dstv)
        plsc.subcore_barrier()

        def gather(j, buf, sem):
            pltpu.async_copy(y.at[srcv.at[j]], buf, sem)

        def gwait(j, buf, sem):
            pltpu.make_async_copy(y.at[srcv.at[j]], buf, sem).wait()

        gather(0, bufa, sga)

        def step(jj, carry):
            j = 2 * jj
            gather(j + 1, bufb, sgb)
            gwait(j, bufa, sga)
            pltpu.sync_copy(bufa, acc.at[dstv.at[j]], add=True)

            @pl.when(jj < CPW // 2 - 1)
            def _():
                gather(j + 2, bufa, sga)

            gwait(j + 1, bufb, sgb)
            pltpu.sync_copy(bufb, acc.at[dstv.at[j + 1]], add=True)
            return carry

        lax.fori_loop(0, CPW // 2, step, 0)
        plsc.subcore_barrier()
        _sliced_copy(acc, agg_out.at[c], s)

    return pl.kernel(
        body,
        out_type=jax.ShapeDtypeStruct((2, N, d), jnp.float32),
        mesh=mesh,
        scratch_types=[
            pltpu.VMEM((CPW, CHUNK), jnp.int32),
            pltpu.VMEM((CPW, CHUNK), jnp.int32),
            pltpu.VMEM((CHUNK, d), jnp.float32),
            pltpu.VMEM((CHUNK, d), jnp.float32),
            pltpu.VMEM_SHARED((NACC, d), jnp.float32),
            pltpu.SemaphoreType.DMA,
            pltpu.SemaphoreType.DMA,
        ])


def _make_sc_deg():
    mesh = plsc.VectorSubcoreMesh(core_axis_name="c", subcore_axis_name="s")

    def body(dsts, zerdeg, ones, deg_out, dstv, onev, dacc):
        c = lax.axis_index("c")
        s = lax.axis_index("s")
        w = c * NSUB + s
        _sliced_copy(zerdeg, dacc, s)
        pltpu.sync_copy(dsts.at[w], dstv)
        pltpu.sync_copy(ones, onev)
        plsc.subcore_barrier()

        def step(j, carry):
            pltpu.sync_copy(onev, dacc.at[dstv.at[j]], add=True)
            return carry

        lax.fori_loop(0, CPW, step, 0)
        plsc.subcore_barrier()
        _sliced_copy(dacc, deg_out.at[c], s)

    return pl.kernel(
        body,
        out_type=jax.ShapeDtypeStruct((2, N, DEGW), jnp.float32),
        mesh=mesh,
        scratch_types=[
            pltpu.VMEM((CPW, CHUNK), jnp.int32),
            pltpu.VMEM((CHUNK, DEGW), jnp.float32),
            pltpu.VMEM_SHARED((NACC, DEGW), jnp.float32),
        ])


def _c1_body(x, w, b, o):
    o[...] = jnp.maximum(
        jnp.dot(x[...], w[...], preferred_element_type=jnp.float32) + b[...],
        0.0)


def _c2_body(agg, deg, h0, wl, bl, wr, z_ref, dinv_ref, ssum, ssq, a1, a2):
    i = pl.program_id(0)
    aggs = agg[0] + agg[1]
    degs = deg[0, :, 0:1] + deg[1, :, 0:1]
    dinv = 1.0 / jnp.maximum(degs, 1.0)
    dinv_ref[...] = jnp.broadcast_to(dinv, (dinv.shape[0], DINVW))
    z = (jnp.dot(aggs * dinv, wl[...], preferred_element_type=jnp.float32)
         + bl[...]
         + jnp.dot(h0[...], wr[...], preferred_element_type=jnp.float32))
    z_ref[...] = z

    @pl.when(i == 0)
    def _():
        a1[...] = jnp.zeros_like(a1)
        a2[...] = jnp.zeros_like(a2)

    a1[...] += jnp.sum(z, axis=0, keepdims=True)
    a2[...] += jnp.sum(z * z, axis=0, keepdims=True)

    @pl.when(i == NB - 1)
    def _():
        ssum[...] = a1[...]
        ssq[...] = a2[...]


def _bn2mm_body(z, s, q, g, b, wl, wr, y_ref, r_ref):
    mu = s[...] * (1.0 / N)
    var = jnp.maximum(q[...] * (1.0 / N) - mu * mu, 0.0)
    rstd = lax.rsqrt(var + EPS)
    h = jnp.maximum((z[...] - mu) * rstd * g[...] + b[...], 0.0)
    y_ref[...] = jnp.dot(h, wl[...], preferred_element_type=jnp.float32)
    r_ref[...] = jnp.dot(h, wr[...], preferred_element_type=jnp.float32)


def _aggadd_body(agg, dinv, r, bl, z_ref, ssum, ssq, a1, a2):
    i = pl.program_id(0)
    z = (agg[0] + agg[1])[:, :r.shape[-1]] * dinv[:, 0:1] + bl[...] + r[...]
    z_ref[...] = z

    @pl.when(i == 0)
    def _():
        a1[...] = jnp.zeros_like(a1)
        a2[...] = jnp.zeros_like(a2)

    a1[...] += jnp.sum(z, axis=0, keepdims=True)
    a2[...] += jnp.sum(z * z, axis=0, keepdims=True)

    @pl.when(i == NB - 1)
    def _():
        ssum[...] = a1[...]
        ssq[...] = a2[...]


def _bnmm_body(z, s, q, g, b, w, wb, z2_ref, osum, osq, a1, a2):
    i = pl.program_id(0)
    mu = s[...] * (1.0 / N)
    var = jnp.maximum(q[...] * (1.0 / N) - mu * mu, 0.0)
    h = jnp.maximum((z[...] - mu) * lax.rsqrt(var + EPS) * g[...] + b[...],
                    0.0)
    z2 = jnp.dot(h, w[...], preferred_element_type=jnp.float32) + wb[...]
    z2_ref[...] = z2

    @pl.when(i == 0)
    def _():
        a1[...] = jnp.zeros_like(a1)
        a2[...] = jnp.zeros_like(a2)

    a1[...] += jnp.sum(z2, axis=0, keepdims=True)
    a2[...] += jnp.sum(z2 * z2, axis=0, keepdims=True)

    @pl.when(i == NB - 1)
    def _():
        osum[...] = a1[...]
        osq[...] = a2[...]


def _c8_body(z, s, q, g, b, w, wb, bt, z2_ref, osum, osq, sums_ref, cnts_ref,
             a1, a2, asu, acn):
    i = pl.program_id(0)
    mu = s[...] * (1.0 / N)
    var = jnp.maximum(q[...] * (1.0 / N) - mu * mu, 0.0)
    h = jnp.maximum((z[...] - mu) * lax.rsqrt(var + EPS) * g[...] + b[...],
                    0.0)
    z2 = jnp.dot(h, w[...], preferred_element_type=jnp.float32) + wb[...]
    z2_ref[...] = z2
    oh = (bt[...] == lax.broadcasted_iota(jnp.int32, (1, G), 1)).astype(
        jnp.float32)

    @pl.when(i == 0)
    def _():
        a1[...] = jnp.zeros_like(a1)
        a2[...] = jnp.zeros_like(a2)
        asu[...] = jnp.zeros_like(asu)
        acn[...] = jnp.zeros_like(acn)

    a1[...] += jnp.sum(z2, axis=0, keepdims=True)
    a2[...] += jnp.sum(z2 * z2, axis=0, keepdims=True)
    asu[...] += lax.dot_general(oh, h, (((0,), (0,)), ((), ())),
                                preferred_element_type=jnp.float32)
    acn[...] += lax.dot_general(oh, jnp.ones((ROWS, 8), jnp.float32),
                                (((0,), (0,)), ((), ())),
                                preferred_element_type=jnp.float32)

    @pl.when(i == NB - 1)
    def _():
        osum[...] = a1[...]
        osq[...] = a2[...]
        sums_ref[...] = asu[...]
        cnts_ref[...] = acn[...]


def _c9_body(z, s, q, g, b, w5, b5, sums, cnts, w4, b4, nout, gout):
    i = pl.program_id(0)
    mu = s[...] * (1.0 / N)
    var = jnp.maximum(q[...] * (1.0 / N) - mu * mu, 0.0)
    h = jnp.maximum((z[...] - mu) * lax.rsqrt(var + EPS) * g[...] + b[...],
                    0.0)
    nout[...] = jnp.dot(h, w5[...], preferred_element_type=jnp.float32) \
        + b5[...]

    @pl.when(i == 0)
    def _():
        gv = sums[...] * (1.0 / jnp.maximum(cnts[...][:, 0:1], 1.0))
        gout[...] = jnp.dot(gv, w4[...],
                            preferred_element_type=jnp.float32) + b4[...]


def kernel(x, fc1_w, fc1_b, sage1_wl, sage1_bl, sage1_wr, bn1_g, bn1_b,
           sage2_wl, sage2_bl, sage2_wr, bn2_g, bn2_b,
           sage3_wl, sage3_bl, sage3_wr, bn3_g, bn3_b,
           fc2_w, fc2_b, bn4_g, bn4_b, fc3_w, fc3_b, bn5_g, bn5_b,
           fc5_w, fc5_b, fc4_w, fc4_b, edge_index, batch):
    f32 = jnp.float32
    row2 = lambda v: v.reshape(1, -1)

    pad = EPAD - E
    src = jnp.concatenate(
        [edge_index[0], jnp.zeros((pad,), jnp.int32)]).reshape(NW, CPW, CHUNK)
    dst = jnp.concatenate(
        [edge_index[1],
         N + (jnp.arange(pad, dtype=jnp.int32) % 8)]).reshape(NW, CPW, CHUNK)
    zer128 = jnp.zeros((N, 128), f32)
    onesdeg = jnp.ones((CHUNK, DEGW), f32)
    batch2 = batch.reshape(N, 1)

    h0 = pl.pallas_call(
        _c1_body, grid=(NB,),
        in_specs=[_rows_spec(256), _full_spec((256, 128)),
                  _full_spec((1, 128))],
        out_specs=_rows_spec(128),
        out_shape=jax.ShapeDtypeStruct((N, 128), f32),
    )(x, fc1_w.T, row2(fc1_b))

    deg = _make_sc_deg()(dst, zer128, onesdeg)
    agg1 = _make_sc_agg(128)(h0, src, dst, zer128)

    z1, dinv, s1, q1 = pl.pallas_call(
        _c2_body, grid=(NB,),
        in_specs=[pl.BlockSpec((2, ROWS, 128), lambda i: (0, i, 0)),
                  pl.BlockSpec((2, ROWS, DEGW), lambda i: (0, i, 0)),
                  _rows_spec(128), _full_spec((128, 256)),
                  _full_spec((1, 256)), _full_spec((128, 256))],
        out_specs=[_rows_spec(256), _rows_spec(DINVW),
                   _full_spec((1, 256)), _full_spec((1, 256))],
        out_shape=[jax.ShapeDtypeStruct((N, 256), f32),
                   jax.ShapeDtypeStruct((N, DINVW), f32),
                   jax.ShapeDtypeStruct((1, 256), f32),
                   jax.ShapeDtypeStruct((1, 256), f32)],
        scratch_shapes=[pltpu.VMEM((1, 256), f32), pltpu.VMEM((1, 256), f32)],
    )(agg1, deg, h0, sage1_wl.T, row2(sage1_bl), sage1_wr.T)

    y2, r2 = pl.pallas_call(
        _bn2mm_body, grid=(NB,),
        in_specs=[_rows_spec(256), _full_spec((1, 256)), _full_spec((1, 256)),
                  _full_spec((1, 256)), _full_spec((1, 256)),
                  _full_spec((256, 128)), _full_spec((256, 128))],
        out_specs=[_rows_spec(128), _rows_spec(128)],
        out_shape=[jax.ShapeDtypeStruct((N, 128), f32),
                   jax.ShapeDtypeStruct((N, 128), f32)],
    )(z1, s1, q1, row2(bn1_g), row2(bn1_b), sage2_wl.T, sage2_wr.T)

    agg2 = _make_sc_agg(128)(y2, src, dst, zer128)

    z2, s2, q2 = pl.pallas_call(
        _aggadd_body, grid=(NB,),
        in_specs=[pl.BlockSpec((2, ROWS, 128), lambda i: (0, i, 0)),
                  _rows_spec(DINVW), _rows_spec(128), _full_spec((1, 128))],
        out_specs=[_rows_spec(128), _full_spec((1, 128)),
                   _full_spec((1, 128))],
        out_shape=[jax.ShapeDtypeStruct((N, 128), f32),
                   jax.ShapeDtypeStruct((1, 128), f32),
                   jax.ShapeDtypeStruct((1, 128), f32)],
        scratch_shapes=[pltpu.VMEM((1, 128), f32), pltpu.VMEM((1, 128), f32)],
    )(agg2, dinv, r2, row2(sage2_bl))

    wl3p = jnp.zeros((128, 128), f32).at[:, :64].set(sage3_wl.T)
    y3, r3 = pl.pallas_call(
        _bn2mm_body, grid=(NB,),
        in_specs=[_rows_spec(128), _full_spec((1, 128)), _full_spec((1, 128)),
                  _full_spec((1, 128)), _full_spec((1, 128)),
                  _full_spec((128, 128)), _full_spec((128, 64))],
        out_specs=[_rows_spec(128), _rows_spec(64)],
        out_shape=[jax.ShapeDtypeStruct((N, 128), f32),
                   jax.ShapeDtypeStruct((N, 64), f32)],
    )(z2, s2, q2, row2(bn2_g), row2(bn2_b), wl3p, sage3_wr.T)

    agg3 = _make_sc_agg(128)(y3, src, dst, zer128)

    z3, s3, q3 = pl.pallas_call(
        _aggadd_body, grid=(NB,),
        in_specs=[pl.BlockSpec((2, ROWS, 128), lambda i: (0, i, 0)),
                  _rows_spec(DINVW), _rows_spec(64), _full_spec((1, 64))],
        out_specs=[_rows_spec(64), _full_spec((1, 64)), _full_spec((1, 64))],
        out_shape=[jax.ShapeDtypeStruct((N, 64), f32),
                   jax.ShapeDtypeStruct((1, 64), f32),
                   jax.ShapeDtypeStruct((1, 64), f32)],
        scratch_shapes=[pltpu.VMEM((1, 64), f32), pltpu.VMEM((1, 64), f32)],
    )(agg3, dinv, r3, row2(sage3_bl))

    z4, s4, q4 = pl.pallas_call(
        _bnmm_body, grid=(NB,),
        in_specs=[_rows_spec(64), _full_spec((1, 64)), _full_spec((1, 64)),
                  _full_spec((1, 64)), _full_spec((1, 64)),
                  _full_spec((64, 64)), _full_spec((1, 64))],
        out_specs=[_rows_spec(64), _full_spec((1, 64)), _full_spec((1, 64))],
        out_shape=[jax.ShapeDtypeStruct((N, 64), f32),
                   jax.ShapeDtypeStruct((1, 64), f32),
                   jax.ShapeDtypeStruct((1, 64), f32)],
        scratch_shapes=[pltpu.VMEM((1, 64), f32), pltpu.VMEM((1, 64), f32)],
    )(z3, s3, q3, row2(bn3_g), row2(bn3_b), fc2_w.T, row2(fc2_b))

    z5, s5, q5, sums, cnts = pl.pallas_call(
        _c8_body, grid=(NB,),
        in_specs=[_rows_spec(64), _full_spec((1, 64)), _full_spec((1, 64)),
                  _full_spec((1, 64)), _full_spec((1, 64)),
                  _full_spec((64, 128)), _full_spec((1, 128)),
                  _rows_spec(1)],
        out_specs=[_rows_spec(128), _full_spec((1, 128)),
                   _full_spec((1, 128)), _full_spec((G, 64)),
                   _full_spec((G, 8))],
        out_shape=[jax.ShapeDtypeStruct((N, 128), f32),
                   jax.ShapeDtypeStruct((1, 128), f32),
                   jax.ShapeDtypeStruct((1, 128), f32),
                   jax.ShapeDtypeStruct((G, 64), f32),
                   jax.ShapeDtypeStruct((G, 8), f32)],
        scratch_shapes=[pltpu.VMEM((1, 128), f32), pltpu.VMEM((1, 128), f32),
                        pltpu.VMEM((G, 64), f32), pltpu.VMEM((G, 8), f32)],
    )(z4, s4, q4, row2(bn4_g), row2(bn4_b), fc3_w.T, row2(fc3_b), batch2)

    w5p = jnp.zeros((128, 128), f32).at[:, :5].set(fc5_w.T)
    b5p = jnp.zeros((1, 128), f32).at[0, :5].set(fc5_b)
    w4p = jnp.zeros((64, 128), f32).at[:, :3].set(fc4_w.T)
    b4p = jnp.zeros((1, 128), f32).at[0, :3].set(fc4_b)
    noutp, goutp = pl.pallas_call(
        _c9_body, grid=(NB,),
        in_specs=[_rows_spec(128), _full_spec((1, 128)), _full_spec((1, 128)),
                  _full_spec((1, 128)), _full_spec((1, 128)),
                  _full_spec((128, 128)), _full_spec((1, 128)),
                  _full_spec((G, 64)), _full_spec((G, 8)),
                  _full_spec((64, 128)), _full_spec((1, 128))],
        out_specs=[_rows_spec(128), _full_spec((G, 128))],
        out_shape=[jax.ShapeDtypeStruct((N, 128), f32),
                   jax.ShapeDtypeStruct((G, 128), f32)],
    )(z5, s5, q5, row2(bn5_g), row2(bn5_b), w5p, b5p, sums, cnts, w4p, b4p)

    return noutp[:, :5], goutp[:, :3]
